# Initial kernel scaffold; baseline (speedup 1.0000x reference)
import jax
import jax.numpy as jnp
from jax.experimental import pallas as pl


def kernel(cost):
    raise NotImplementedError("write your pallas kernel here")


# speedup vs baseline: 18.1553x; 18.1553x over previous
import functools

import jax
import jax.numpy as jnp
from jax.experimental import pallas as pl

D1 = 48
D2 = 48
K = 3


def _tree_gather(rows, idx):
    c = rows.shape[0]
    pad = 64 - c
    if pad:
        rows = jnp.concatenate([rows, jnp.zeros((pad,) + rows.shape[1:], rows.dtype)], axis=0)
    cur = rows[:, None]
    for bit in range(6):
        n = cur.shape[0]
        cur = cur.reshape((n // 2, 2) + cur.shape[1:])
        mask = ((idx >> bit) & 1).astype(jnp.bool_)[None]
        cur = jnp.where(mask, cur[:, 1], cur[:, 0])
    return cur[0]


def _body(cost_ref, out_ref):
    shp = cost_ref.shape[1:]
    neg = jnp.full(shp, -jnp.inf, jnp.float32)
    zero_i = jnp.zeros(shp, jnp.int32)

    def step(i, carry):
        v0, v1, v2, i0, i1, i2 = carry
        x = cost_ref[i]
        ix = jnp.full(shp, i, jnp.int32)
        b0 = x >= v0
        nv0 = jnp.maximum(v0, x)
        dx = jnp.minimum(v0, x)
        ni0 = jnp.where(b0, ix, i0)
        di = jnp.where(b0, i0, ix)
        b1 = dx >= v1
        nv1 = jnp.maximum(v1, dx)
        dx2 = jnp.minimum(v1, dx)
        ni1 = jnp.where(b1, di, i1)
        di2 = jnp.where(b1, i1, di)
        b2 = dx2 >= v2
        nv2 = jnp.maximum(v2, dx2)
        ni2 = jnp.where(b2, di2, i2)
        return nv0, nv1, nv2, ni0, ni1, ni2

    v0, v1, v2, i0, i1, i2 = jax.lax.fori_loop(
        0, D1, step, (neg, neg, neg, zero_i, zero_i, zero_i))

    cv0 = _tree_gather(cost_ref[0], i0)
    cv1 = _tree_gather(cost_ref[1], i1)
    cv2 = _tree_gather(cost_ref[2], i2)

    m = jnp.maximum(cv0, jnp.maximum(cv1, cv2))
    e0 = jnp.exp(cv0 - m)
    e1 = jnp.exp(cv1 - m)
    e2 = jnp.exp(cv2 - m)
    inv = 1.0 / (e0 + e1 + e2)
    out_ref[...] = (e0 * i0.astype(jnp.float32)
                    + e1 * i1.astype(jnp.float32)
                    + e2 * i2.astype(jnp.float32)) * inv


@functools.partial(jax.jit, static_argnames=("interpret",))
def _run(cost, interpret=False):
    b, d1, d2, h, w = cost.shape
    c = cost.reshape(d1, d2, h, w)
    ht = 8
    grid = (h // ht,)
    out = pl.pallas_call(
        _body,
        grid=grid,
        in_specs=[pl.BlockSpec((d1, d2, ht, w), lambda g: (0, 0, g, 0))],
        out_specs=pl.BlockSpec((d2, ht, w), lambda g: (0, g, 0)),
        out_shape=jax.ShapeDtypeStruct((d2, h, w), jnp.float32),
        interpret=interpret,
    )(c)
    return out.reshape(b, 1, d2, h, w)


def kernel(cost):
    return _run(cost)


# speedup vs baseline: 56.1344x; 3.0919x over previous
import functools

import jax
import jax.numpy as jnp
from jax.experimental import pallas as pl

D1 = 48
D2 = 48
JC = 8


def _tree_gather(rows, idx):
    lo_bits = [((idx >> b) & 1).astype(jnp.bool_) for b in range(4)]
    groups = []
    for g in range(3):
        cur = [r[None] for r in rows[16 * g:16 * (g + 1)]]
        for b in range(4):
            cur = [jnp.where(lo_bits[b], cur[2 * t + 1], cur[2 * t])
                   for t in range(len(cur) // 2)]
        groups.append(cur[0])
    hi0 = (idx >> 4) & 3
    out = jnp.where(hi0 == 1, groups[1], groups[0])
    return jnp.where(hi0 == 2, groups[2], out)


def _body(cost_ref, out_ref):
    ht, w = cost_ref.shape[2], cost_ref.shape[3]
    for c0 in range(0, D2, JC):
        shp = (JC, ht, w)
        neg = jnp.full(shp, -jnp.inf, jnp.float32)
        zero_i = jnp.zeros(shp, jnp.int32)
        v0 = v1 = v2 = neg
        i0 = i1 = i2 = zero_i
        for i in range(D1):
            x = cost_ref[i, c0:c0 + JC]
            ix = jnp.full(shp, i, jnp.int32)
            b0 = x >= v0
            nv0 = jnp.maximum(v0, x)
            dx = jnp.minimum(v0, x)
            ni0 = jnp.where(b0, ix, i0)
            di = jnp.where(b0, i0, ix)
            b1 = dx >= v1
            nv1 = jnp.maximum(v1, dx)
            dx2 = jnp.minimum(v1, dx)
            ni1 = jnp.where(b1, di, i1)
            di2 = jnp.where(b1, i1, di)
            b2 = dx2 >= v2
            v2 = jnp.maximum(v2, dx2)
            i2 = jnp.where(b2, di2, i2)
            v0, v1, i0, i1 = nv0, nv1, ni0, ni1

        rows0 = [cost_ref[0, c] for c in range(D1)]
        rows1 = [cost_ref[1, c] for c in range(D1)]
        rows2 = [cost_ref[2, c] for c in range(D1)]
        cv0 = _tree_gather(rows0, i0)
        cv1 = _tree_gather(rows1, i1)
        cv2 = _tree_gather(rows2, i2)

        m = jnp.maximum(cv0, jnp.maximum(cv1, cv2))
        e0 = jnp.exp(cv0 - m)
        e1 = jnp.exp(cv1 - m)
        e2 = jnp.exp(cv2 - m)
        inv = 1.0 / (e0 + e1 + e2)
        out_ref[c0:c0 + JC] = (e0 * i0.astype(jnp.float32)
                               + e1 * i1.astype(jnp.float32)
                               + e2 * i2.astype(jnp.float32)) * inv


@functools.partial(jax.jit, static_argnames=("interpret",))
def _run(cost, interpret=False):
    b, d1, d2, h, w = cost.shape
    c = cost.reshape(d1, d2, h, w)
    ht = 8
    grid = (h // ht,)
    out = pl.pallas_call(
        _body,
        grid=grid,
        in_specs=[pl.BlockSpec((d1, d2, ht, w), lambda g: (0, 0, g, 0))],
        out_specs=pl.BlockSpec((d2, ht, w), lambda g: (0, g, 0)),
        out_shape=jax.ShapeDtypeStruct((d2, h, w), jnp.float32),
        interpret=interpret,
    )(c)
    return out.reshape(b, 1, d2, h, w)


def kernel(cost):
    return _run(cost)
